# baseline (speedup 1.0000x reference)
import functools

import jax
import jax.numpy as jnp
from jax import lax
from jax.experimental import pallas as pl
from jax.experimental.pallas import tpu as pltpu
from jax.experimental.pallas import tpu_sc as plsc

B = 8
N = 8192
S = 2048
K = 32
R2 = 0.25 * 0.25
EPS = 1e-5
TOT = B * S * K

NTILES = 32
SC_CHUNK = S // 4
GROWS = (B * S * K) // NTILES
GCHUNK = 2048


NCK = N // 128


def _fps_body(xyz_ref, nxyz_ref, dist_ref):
    x = xyz_ref[0]
    y = xyz_ref[1]
    z = xyz_ref[2]
    lane = lax.broadcasted_iota(jnp.int32, (B, 128), 1)
    bigi = jnp.int32(N)
    dist_ref[...] = jnp.full((B, N), 1e10, dtype=jnp.float32)

    def body(i, carry):
        cx, cy, cz, rx, ry, rz = carry
        lsel = lane == lax.rem(i, 128)
        rx = jnp.where(lsel, cx, rx)
        ry = jnp.where(lsel, cy, ry)
        rz = jnp.where(lsel, cz, rz)

        @pl.when(lax.rem(i, 128) == 127)
        def _():
            j = lax.div(i, 128)
            nxyz_ref[pl.ds(j, 1), :, 0, :] = rx[None]
            nxyz_ref[pl.ds(j, 1), :, 1, :] = ry[None]
            nxyz_ref[pl.ds(j, 1), :, 2, :] = rz[None]

        accs = [None] * 4
        for k in range(NCK):
            sl = slice(128 * k, 128 * (k + 1))
            xk = x[:, sl]
            yk = y[:, sl]
            zk = z[:, sl]
            dxk = xk - cx
            dyk = yk - cy
            dzk = zk - cz
            dk = dxk * dxk + dyk * dyk + dzk * dzk
            ndk = jnp.minimum(dist_ref[:, sl], dk)
            dist_ref[:, sl] = ndk
            kc = jnp.full((B, 128), k, jnp.int32)
            a = accs[k % 4]
            if a is None:
                accs[k % 4] = (ndk, xk, yk, zk, kc)
            else:
                gt = ndk > a[0]
                accs[k % 4] = tuple(
                    jnp.where(gt, u, v)
                    for v, u in zip(a, (ndk, xk, yk, zk, kc)))

        def merge(a, bb):
            take = jnp.logical_or(
                bb[0] > a[0],
                jnp.logical_and(bb[0] == a[0], bb[4] < a[4]))
            return tuple(jnp.where(take, u, v) for v, u in zip(a, bb))

        bv, bx, by, bz, bk = merge(merge(accs[0], accs[1]),
                                   merge(accs[2], accs[3]))
        m = jnp.max(bv, axis=1, keepdims=True)
        lidx = jnp.where(bv == m, bk * 128 + lane, bigi)
        nmin = jnp.min(lidx, axis=1, keepdims=True)
        hot = lidx == nmin
        cx = jnp.sum(jnp.where(hot, bx, 0.0), axis=1, keepdims=True)
        cy = jnp.sum(jnp.where(hot, by, 0.0), axis=1, keepdims=True)
        cz = jnp.sum(jnp.where(hot, bz, 0.0), axis=1, keepdims=True)
        return cx, cy, cz, rx, ry, rz

    cx0 = x[:, 0:1]
    cy0 = y[:, 0:1]
    cz0 = z[:, 0:1]
    r0 = jnp.zeros((B, 128), dtype=jnp.float32)
    lax.fori_loop(0, S, body, (cx0, cy0, cz0, r0, r0, r0))


def _fps(xyz3):
    return pl.pallas_call(
        _fps_body,
        out_shape=jax.ShapeDtypeStruct((S // 128, B, 3, 128), jnp.float32),
        scratch_shapes=[pltpu.VMEM((B, N), jnp.float32)],
    )(xyz3)


def _gtable_body(xyz_ref, pts_ref, nxyz_ref, w1a_ref, w1b_ref, b1_ref,
                 g_ref, cb_ref):
    dn = (((0,), (1,)), ((), ()))
    g = lax.dot_general(xyz_ref[0], w1a_ref[...], dn,
                        preferred_element_type=jnp.float32)
    g = g + lax.dot_general(pts_ref[0], w1b_ref[...], dn,
                            preferred_element_type=jnp.float32)
    g_ref[0] = g
    cb = lax.dot_general(nxyz_ref[0], w1a_ref[...], dn,
                         preferred_element_type=jnp.float32)
    cb_ref[0] = cb - b1_ref[...]


def _gtable(xyz, points, new_xyz, w1a, w1b, b1r):
    return pl.pallas_call(
        _gtable_body,
        grid=(B,),
        in_specs=[
            pl.BlockSpec((1, 3, N), lambda b: (b, 0, 0)),
            pl.BlockSpec((1, 32, N), lambda b: (b, 0, 0)),
            pl.BlockSpec((1, 3, S), lambda b: (b, 0, 0)),
            pl.BlockSpec((32, 3), lambda b: (0, 0)),
            pl.BlockSpec((32, 32), lambda b: (0, 0)),
            pl.BlockSpec((1, 32), lambda b: (0, 0)),
        ],
        out_specs=[
            pl.BlockSpec((1, N, 32), lambda b: (b, 0, 0)),
            pl.BlockSpec((1, S, 32), lambda b: (b, 0, 0)),
        ],
        out_shape=[
            jax.ShapeDtypeStruct((B, N, 32), jnp.float32),
            jax.ShapeDtypeStruct((B, S, 32), jnp.float32),
        ],
    )(xyz, points, new_xyz, w1a, w1b, b1r)


def _ballq_body(xyz_hbm, nxyz_hbm, out_hbm, xv, yv, zv, qxv, qyv, qzv, obuf):
    wid = lax.axis_index("s") * 2 + lax.axis_index("c")
    b = wid // 4
    chunk = wid % 4
    base_s = chunk * SC_CHUNK
    pltpu.sync_copy(xyz_hbm.at[pl.ds((b * 3 + 0) * N, N)], xv)
    pltpu.sync_copy(xyz_hbm.at[pl.ds((b * 3 + 1) * N, N)], yv)
    pltpu.sync_copy(xyz_hbm.at[pl.ds((b * 3 + 2) * N, N)], zv)
    pltpu.sync_copy(nxyz_hbm.at[pl.ds((b * 3 + 0) * S + base_s, SC_CHUNK)], qxv)
    pltpu.sync_copy(nxyz_hbm.at[pl.ds((b * 3 + 1) * S + base_s, SC_CHUNK)], qyv)
    pltpu.sync_copy(nxyz_hbm.at[pl.ds((b * 3 + 2) * S + base_s, SC_CHUNK)], qzv)
    lane = lax.iota(jnp.int32, 16)
    gbase = b * N

    def per_centroid(c, _):
        ci = jnp.full((16,), c, jnp.int32)
        qx = plsc.load_gather(qxv, [ci])
        qy = plsc.load_gather(qyv, [ci])
        qz = plsc.load_gather(qzv, [ci])

        U = 4

        def cond(carry):
            n, count = carry
            return jnp.logical_and(count < K, n < N // (16 * U))

        def body(carry):
            n, count = carry
            for u in range(U):
                base = (n * U + u) * 16
                px = xv[pl.ds(base, 16)]
                py = yv[pl.ds(base, 16)]
                pz = zv[pl.ds(base, 16)]
                dx = qx - px
                dy = qy - py
                dz = qz - pz
                d = dx * dx + dy * dy + dz * dz
                pred = (d < R2).astype(jnp.int32)
                pi = plsc.cumsum(pred)
                pos = count + pi - 1
                m = jnp.logical_and(pred > 0, pos < K)
                plsc.store_scatter(obuf, [c * K + pos],
                                   gbase + base + lane, mask=m)
                count = count + pi[15]
            return n + 1, count

        _, cnt = lax.while_loop(cond, body, (jnp.int32(0), jnp.int32(0)))
        cnt = jnp.minimum(cnt, K)
        first = plsc.load_gather(obuf, [jnp.full((16,), c * K, jnp.int32)])
        for h in range(K // 16):
            j = lane + h * 16
            cur = obuf[pl.ds(c * K + h * 16, 16)]
            obuf[pl.ds(c * K + h * 16, 16)] = jnp.where(j < cnt, cur, first)
        return 0

    lax.fori_loop(0, SC_CHUNK, per_centroid, 0)
    pltpu.sync_copy(obuf,
                    out_hbm.at[pl.ds(b * S * K + base_s * K, SC_CHUNK * K)])


def _ballq(xyz, new_xyz):
    mesh = plsc.VectorSubcoreMesh(core_axis_name="c", subcore_axis_name="s")
    f = functools.partial(
        pl.kernel, mesh=mesh,
        compiler_params=pltpu.CompilerParams(needs_layout_passes=False),
        out_type=jax.ShapeDtypeStruct((B * S * K,), jnp.int32),
        scratch_types=[
            pltpu.VMEM((N,), jnp.float32),
            pltpu.VMEM((N,), jnp.float32),
            pltpu.VMEM((N,), jnp.float32),
            pltpu.VMEM((SC_CHUNK,), jnp.float32),
            pltpu.VMEM((SC_CHUNK,), jnp.float32),
            pltpu.VMEM((SC_CHUNK,), jnp.float32),
            pltpu.VMEM((SC_CHUNK * K,), jnp.int32),
        ],
    )(_ballq_body)
    return f(xyz, new_xyz)


def _grows_body(g_hbm, idx_hbm, out_hbm, idxv, rowsv, sem):
    wid = lax.axis_index("s") * 2 + lax.axis_index("c")
    base = wid * GROWS

    def chunk_body(ci, _):
        off = base + ci * GCHUNK
        pltpu.sync_copy(idx_hbm.at[pl.ds(off, GCHUNK)], idxv)
        pltpu.async_copy(g_hbm.at[idxv], rowsv, sem).wait()
        pltpu.sync_copy(rowsv, out_hbm.at[pl.ds(off, GCHUNK)])
        return 0

    lax.fori_loop(0, GROWS // GCHUNK, chunk_body, 0)


def _grows(g_flat, idx_flat):
    mesh = plsc.VectorSubcoreMesh(core_axis_name="c", subcore_axis_name="s")
    f = functools.partial(
        pl.kernel, mesh=mesh,
        compiler_params=pltpu.CompilerParams(use_tc_tiling_on_sc=False),
        out_type=jax.ShapeDtypeStruct((B * S * K, 32), jnp.float32),
        scratch_types=[
            pltpu.VMEM((GCHUNK,), jnp.int32),
            pltpu.VMEM((GCHUNK, 32), jnp.float32),
            pltpu.SemaphoreType.DMA,
        ],
    )(_grows_body)
    return f(g_flat, idx_flat)


CH = 4096
NCHUNK = TOT // CH
CCB = CH // K


CH4 = CH // 4


def _mlp_body(gg_ref, cb_ref, w2_ref, b2_ref, w3_ref, b3_ref,
              g1_ref, be1_ref, g2_ref, be2_ref, g3_ref, be3_ref,
              out_ref, st_ref):
    p = pl.program_id(0)
    t = pl.program_id(1)
    dn = (((1,), (0,)), ((), ()))
    inv = jnp.float32(1.0 / TOT)

    @pl.when(jnp.logical_and(p == 0, t == 0))
    def _():
        st_ref[...] = jnp.zeros((16, 256), jnp.float32)

    def lanefold(row, width, lanes, op):
        a = st_ref[row:row + 1, :lanes]
        r = a[:, 0:width]
        for j in range(1, lanes // width):
            r = op(r, a[:, j * width:(j + 1) * width])
        return r

    def finalize(src, dst, width, lanes):
        nb = lanes // width
        s = lanefold(src, width, lanes, jnp.add)
        q = lanefold(src + 1, width, lanes, jnp.add)
        mean = s * inv
        var = q * inv - mean * mean
        rs = lax.rsqrt(var + EPS)
        st_ref[dst:dst + 1, :lanes] = jnp.concatenate([mean] * nb, axis=1)
        st_ref[dst + 1:dst + 2, :lanes] = jnp.concatenate([rs] * nb, axis=1)

    @pl.when(jnp.logical_and(p == 1, t == 0))
    def _():
        finalize(0, 8, 32, 128)

    @pl.when(jnp.logical_and(p == 2, t == 0))
    def _():
        finalize(2, 10, 32, 128)

    @pl.when(jnp.logical_and(p == 3, t == 0))
    def _():
        finalize(4, 12, 64, 256)

    def y1():
        cb4 = jnp.concatenate([cb_ref[...]] * 4, axis=1)
        cbp = jnp.broadcast_to(cb4[:, None, :], (CCB, 8, 128))
        return gg_ref[...] - cbp.reshape(CH4, 128)

    def norm(yv, dst, grow, berow, lanes):
        xh = (yv - st_ref[dst:dst + 1, :lanes]) * st_ref[dst + 1:dst + 2,
                                                         :lanes]
        return jnp.maximum(xh * grow + berow, 0.0)

    def acc(yv, row, lanes):
        st_ref[row:row + 1, :lanes] += jnp.sum(yv, axis=0, keepdims=True)
        st_ref[row + 1:row + 2, :lanes] += jnp.sum(yv * yv, axis=0,
                                                   keepdims=True)

    @pl.when(p == 0)
    def _():
        acc(y1(), 0, 128)

    @pl.when(p == 1)
    def _():
        x1 = norm(y1(), 8, g1_ref[...], be1_ref[...], 128)
        y2 = lax.dot_general(x1, w2_ref[...], dn,
                             preferred_element_type=jnp.float32) + b2_ref[...]
        acc(y2, 2, 128)

    @pl.when(p == 2)
    def _():
        x1 = norm(y1(), 8, g1_ref[...], be1_ref[...], 128)
        y2 = lax.dot_general(x1, w2_ref[...], dn,
                             preferred_element_type=jnp.float32) + b2_ref[...]
        x2 = norm(y2, 10, g2_ref[...], be2_ref[...], 128)
        y3 = lax.dot_general(x2, w3_ref[...], dn,
                             preferred_element_type=jnp.float32) + b3_ref[...]
        acc(y3, 4, 256)

    @pl.when(p == 3)
    def _():
        x1 = norm(y1(), 8, g1_ref[...], be1_ref[...], 128)
        y2 = lax.dot_general(x1, w2_ref[...], dn,
                             preferred_element_type=jnp.float32) + b2_ref[...]
        x2 = norm(y2, 10, g2_ref[...], be2_ref[...], 128)
        y3 = lax.dot_general(x2, w3_ref[...], dn,
                             preferred_element_type=jnp.float32) + b3_ref[...]
        x3 = norm(y3, 12, g3_ref[...], be3_ref[...], 256)
        m8 = jnp.max(x3.reshape(CCB, 8, 256), axis=1)
        out_ref[...] = jnp.maximum(
            jnp.maximum(m8[:, 0:64], m8[:, 64:128]),
            jnp.maximum(m8[:, 128:192], m8[:, 192:256]))


def _mlp(ggp, cb_flat, w2b, b2p, w3b, b3p, g1p, be1p, g2p, be2p, g3p, be3p):
    wspec = lambda shp: pl.BlockSpec(shp, lambda p, t: (0, 0))
    return pl.pallas_call(
        _mlp_body,
        grid=(4, NCHUNK),
        in_specs=[
            pl.BlockSpec((CH4, 128), lambda p, t: (t, 0)),
            pl.BlockSpec((CCB, 32), lambda p, t: (t, 0)),
            wspec((128, 128)), wspec((1, 128)),
            wspec((128, 256)), wspec((1, 256)),
            wspec((1, 128)), wspec((1, 128)),
            wspec((1, 128)), wspec((1, 128)),
            wspec((1, 256)), wspec((1, 256)),
        ],
        out_specs=pl.BlockSpec((CCB, 64), lambda p, t: (t, 0)),
        out_shape=jax.ShapeDtypeStruct((B * S, 64), jnp.float32),
        scratch_shapes=[pltpu.VMEM((16, 256), jnp.float32)],
    )(ggp, cb_flat, w2b, b2p, w3b, b3p, g1p, be1p, g2p, be2p, g3p, be3p)


def kernel(xyz, points, W1, b1, gamma1, beta1, W2, b2, gamma2, beta2,
           W3, b3, gamma3, beta3):
    fps_raw = _fps(jnp.transpose(xyz, (1, 0, 2)))
    new_xyz = jnp.transpose(fps_raw, (1, 2, 0, 3)).reshape(B, 3, S)
    g, cb = _gtable(xyz, points, new_xyz, W1[:, :3], W1[:, 3:],
                    b1.reshape(1, 32))
    ball = _ballq(xyz.reshape(B * 3 * N), new_xyz.reshape(B * 3 * S))
    rows = _grows(g.reshape(B * N, 32), ball)
    eye4 = jnp.eye(4, dtype=jnp.float32)
    out_sp = _mlp(rows.reshape(B * S * K // 4, 128),
                  cb.reshape(B * S, 32),
                  jnp.kron(eye4, W2.T), jnp.tile(b2, 4).reshape(1, 128),
                  jnp.kron(eye4, W3.T), jnp.tile(b3, 4).reshape(1, 256),
                  jnp.tile(gamma1, 4).reshape(1, 128),
                  jnp.tile(beta1, 4).reshape(1, 128),
                  jnp.tile(gamma2, 4).reshape(1, 128),
                  jnp.tile(beta2, 4).reshape(1, 128),
                  jnp.tile(gamma3, 4).reshape(1, 256),
                  jnp.tile(beta3, 4).reshape(1, 256))
    new_points = jnp.transpose(out_sp.reshape(B, S, 64), (0, 2, 1))
    return (new_xyz, new_points)


# speedup vs baseline: 1.0112x; 1.0112x over previous
import functools

import jax
import jax.numpy as jnp
from jax import lax
from jax.experimental import pallas as pl
from jax.experimental.pallas import tpu as pltpu
from jax.experimental.pallas import tpu_sc as plsc

B = 8
N = 8192
S = 2048
K = 32
R2 = 0.25 * 0.25
EPS = 1e-5
TOT = B * S * K

NTILES = 32
SC_CHUNK = S // 4
GROWS = (B * S * K) // NTILES
GCHUNK = 2048


NCK = N // 128


def _fps_body(xyz_ref, nxyz_ref, dist_ref):
    x = xyz_ref[0]
    y = xyz_ref[1]
    z = xyz_ref[2]
    lane = lax.broadcasted_iota(jnp.int32, (B, 128), 1)
    bigi = jnp.int32(N)
    dist_ref[...] = jnp.full((B, N), 1e10, dtype=jnp.float32)

    def body(i, carry):
        cx, cy, cz, rx, ry, rz = carry
        lsel = lane == lax.rem(i, 128)
        rx = jnp.where(lsel, cx, rx)
        ry = jnp.where(lsel, cy, ry)
        rz = jnp.where(lsel, cz, rz)

        @pl.when(lax.rem(i, 128) == 127)
        def _():
            j = lax.div(i, 128)
            nxyz_ref[pl.ds(j, 1), :, 0, :] = rx[None]
            nxyz_ref[pl.ds(j, 1), :, 1, :] = ry[None]
            nxyz_ref[pl.ds(j, 1), :, 2, :] = rz[None]

        bv = bx = by = bz = bk = None
        for k in range(NCK):
            sl = slice(128 * k, 128 * (k + 1))
            xk = x[:, sl]
            yk = y[:, sl]
            zk = z[:, sl]
            dxk = xk - cx
            dyk = yk - cy
            dzk = zk - cz
            dk = dxk * dxk + dyk * dyk + dzk * dzk
            ndk = jnp.minimum(dist_ref[:, sl], dk)
            dist_ref[:, sl] = ndk
            if k == 0:
                bv, bx, by, bz = ndk, xk, yk, zk
                bk = jnp.zeros((B, 128), jnp.int32)
            else:
                gt = ndk > bv
                bv = jnp.where(gt, ndk, bv)
                bx = jnp.where(gt, xk, bx)
                by = jnp.where(gt, yk, by)
                bz = jnp.where(gt, zk, bz)
                bk = jnp.where(gt, k, bk)
        m = jnp.max(bv, axis=1, keepdims=True)
        lidx = jnp.where(bv == m, bk * 128 + lane, bigi)
        nmin = jnp.min(lidx, axis=1, keepdims=True)
        hot = lidx == nmin
        cx = jnp.sum(jnp.where(hot, bx, 0.0), axis=1, keepdims=True)
        cy = jnp.sum(jnp.where(hot, by, 0.0), axis=1, keepdims=True)
        cz = jnp.sum(jnp.where(hot, bz, 0.0), axis=1, keepdims=True)
        return cx, cy, cz, rx, ry, rz

    cx0 = x[:, 0:1]
    cy0 = y[:, 0:1]
    cz0 = z[:, 0:1]
    r0 = jnp.zeros((B, 128), dtype=jnp.float32)
    lax.fori_loop(0, S, body, (cx0, cy0, cz0, r0, r0, r0))


def _fps(xyz3):
    return pl.pallas_call(
        _fps_body,
        out_shape=jax.ShapeDtypeStruct((S // 128, B, 3, 128), jnp.float32),
        scratch_shapes=[pltpu.VMEM((B, N), jnp.float32)],
    )(xyz3)


def _gtable_body(xyz_ref, pts_ref, nxyz_ref, w1a_ref, w1b_ref, b1_ref,
                 g_ref, cb_ref):
    dn = (((0,), (1,)), ((), ()))
    g = lax.dot_general(xyz_ref[0], w1a_ref[...], dn,
                        preferred_element_type=jnp.float32)
    g = g + lax.dot_general(pts_ref[0], w1b_ref[...], dn,
                            preferred_element_type=jnp.float32)
    g_ref[0] = g
    cb = lax.dot_general(nxyz_ref[0], w1a_ref[...], dn,
                         preferred_element_type=jnp.float32)
    cb_ref[0] = cb - b1_ref[...]


def _gtable(xyz, points, new_xyz, w1a, w1b, b1r):
    return pl.pallas_call(
        _gtable_body,
        grid=(B,),
        in_specs=[
            pl.BlockSpec((1, 3, N), lambda b: (b, 0, 0)),
            pl.BlockSpec((1, 32, N), lambda b: (b, 0, 0)),
            pl.BlockSpec((1, 3, S), lambda b: (b, 0, 0)),
            pl.BlockSpec((32, 3), lambda b: (0, 0)),
            pl.BlockSpec((32, 32), lambda b: (0, 0)),
            pl.BlockSpec((1, 32), lambda b: (0, 0)),
        ],
        out_specs=[
            pl.BlockSpec((1, N, 32), lambda b: (b, 0, 0)),
            pl.BlockSpec((1, S, 32), lambda b: (b, 0, 0)),
        ],
        out_shape=[
            jax.ShapeDtypeStruct((B, N, 32), jnp.float32),
            jax.ShapeDtypeStruct((B, S, 32), jnp.float32),
        ],
    )(xyz, points, new_xyz, w1a, w1b, b1r)


def _ballq_body(xyz_hbm, nxyz_hbm, out_hbm, xv, yv, zv, qxv, qyv, qzv, obuf):
    wid = lax.axis_index("s") * 2 + lax.axis_index("c")
    b = wid // 4
    chunk = wid % 4
    base_s = chunk * SC_CHUNK
    pltpu.sync_copy(xyz_hbm.at[pl.ds((b * 3 + 0) * N, N)], xv)
    pltpu.sync_copy(xyz_hbm.at[pl.ds((b * 3 + 1) * N, N)], yv)
    pltpu.sync_copy(xyz_hbm.at[pl.ds((b * 3 + 2) * N, N)], zv)
    pltpu.sync_copy(nxyz_hbm.at[pl.ds((b * 3 + 0) * S + base_s, SC_CHUNK)], qxv)
    pltpu.sync_copy(nxyz_hbm.at[pl.ds((b * 3 + 1) * S + base_s, SC_CHUNK)], qyv)
    pltpu.sync_copy(nxyz_hbm.at[pl.ds((b * 3 + 2) * S + base_s, SC_CHUNK)], qzv)
    lane = lax.iota(jnp.int32, 16)
    gbase = b * N

    def per_centroid(c, _):
        ci = jnp.full((16,), c, jnp.int32)
        qx = plsc.load_gather(qxv, [ci])
        qy = plsc.load_gather(qyv, [ci])
        qz = plsc.load_gather(qzv, [ci])

        U = 4

        def cond(carry):
            n, count = carry
            return jnp.logical_and(count < K, n < N // (16 * U))

        def body(carry):
            n, count = carry
            for u in range(U):
                base = (n * U + u) * 16
                px = xv[pl.ds(base, 16)]
                py = yv[pl.ds(base, 16)]
                pz = zv[pl.ds(base, 16)]
                dx = qx - px
                dy = qy - py
                dz = qz - pz
                d = dx * dx + dy * dy + dz * dz
                pred = (d < R2).astype(jnp.int32)
                pi = plsc.cumsum(pred)
                pos = count + pi - 1
                m = jnp.logical_and(pred > 0, pos < K)
                plsc.store_scatter(obuf, [c * K + pos],
                                   gbase + base + lane, mask=m)
                count = count + pi[15]
            return n + 1, count

        _, cnt = lax.while_loop(cond, body, (jnp.int32(0), jnp.int32(0)))
        cnt = jnp.minimum(cnt, K)
        first = plsc.load_gather(obuf, [jnp.full((16,), c * K, jnp.int32)])
        for h in range(K // 16):
            j = lane + h * 16
            cur = obuf[pl.ds(c * K + h * 16, 16)]
            obuf[pl.ds(c * K + h * 16, 16)] = jnp.where(j < cnt, cur, first)
        return 0

    lax.fori_loop(0, SC_CHUNK, per_centroid, 0)
    pltpu.sync_copy(obuf,
                    out_hbm.at[pl.ds(b * S * K + base_s * K, SC_CHUNK * K)])


def _ballq(xyz, new_xyz):
    mesh = plsc.VectorSubcoreMesh(core_axis_name="c", subcore_axis_name="s")
    f = functools.partial(
        pl.kernel, mesh=mesh,
        compiler_params=pltpu.CompilerParams(needs_layout_passes=False),
        out_type=jax.ShapeDtypeStruct((B * S * K,), jnp.int32),
        scratch_types=[
            pltpu.VMEM((N,), jnp.float32),
            pltpu.VMEM((N,), jnp.float32),
            pltpu.VMEM((N,), jnp.float32),
            pltpu.VMEM((SC_CHUNK,), jnp.float32),
            pltpu.VMEM((SC_CHUNK,), jnp.float32),
            pltpu.VMEM((SC_CHUNK,), jnp.float32),
            pltpu.VMEM((SC_CHUNK * K,), jnp.int32),
        ],
    )(_ballq_body)
    return f(xyz, new_xyz)


def _grows_body(g_hbm, idx_hbm, out_hbm, idxv, rowsv, sem):
    wid = lax.axis_index("s") * 2 + lax.axis_index("c")
    base = wid * GROWS

    def chunk_body(ci, _):
        off = base + ci * GCHUNK
        pltpu.sync_copy(idx_hbm.at[pl.ds(off, GCHUNK)], idxv)
        pltpu.async_copy(g_hbm.at[idxv], rowsv, sem).wait()
        pltpu.sync_copy(rowsv, out_hbm.at[pl.ds(off, GCHUNK)])
        return 0

    lax.fori_loop(0, GROWS // GCHUNK, chunk_body, 0)


def _grows(g_flat, idx_flat):
    mesh = plsc.VectorSubcoreMesh(core_axis_name="c", subcore_axis_name="s")
    f = functools.partial(
        pl.kernel, mesh=mesh,
        compiler_params=pltpu.CompilerParams(use_tc_tiling_on_sc=False),
        out_type=jax.ShapeDtypeStruct((B * S * K, 32), jnp.float32),
        scratch_types=[
            pltpu.VMEM((GCHUNK,), jnp.int32),
            pltpu.VMEM((GCHUNK, 32), jnp.float32),
            pltpu.SemaphoreType.DMA,
        ],
    )(_grows_body)
    return f(g_flat, idx_flat)


CH = 4096
NCHUNK = TOT // CH
CCB = CH // K


CH4 = CH // 4


def _mlp_body(gg_ref, cb_ref, w2_ref, b2_ref, w3_ref, b3_ref,
              g1_ref, be1_ref, g2_ref, be2_ref, g3_ref, be3_ref,
              out_ref, st_ref):
    p = pl.program_id(0)
    t = pl.program_id(1)
    dn = (((1,), (0,)), ((), ()))
    inv = jnp.float32(1.0 / TOT)

    @pl.when(jnp.logical_and(p == 0, t == 0))
    def _():
        st_ref[...] = jnp.zeros((16, 256), jnp.float32)

    def lanefold(row, width, lanes, op):
        a = st_ref[row:row + 1, :lanes]
        r = a[:, 0:width]
        for j in range(1, lanes // width):
            r = op(r, a[:, j * width:(j + 1) * width])
        return r

    def finalize(src, dst, width, lanes):
        nb = lanes // width
        s = lanefold(src, width, lanes, jnp.add)
        q = lanefold(src + 1, width, lanes, jnp.add)
        mean = s * inv
        var = q * inv - mean * mean
        rs = lax.rsqrt(var + EPS)
        st_ref[dst:dst + 1, :lanes] = jnp.concatenate([mean] * nb, axis=1)
        st_ref[dst + 1:dst + 2, :lanes] = jnp.concatenate([rs] * nb, axis=1)

    @pl.when(jnp.logical_and(p == 1, t == 0))
    def _():
        finalize(0, 8, 32, 128)

    @pl.when(jnp.logical_and(p == 2, t == 0))
    def _():
        finalize(2, 10, 32, 128)

    @pl.when(jnp.logical_and(p == 3, t == 0))
    def _():
        finalize(4, 12, 64, 256)

    def y1():
        cb4 = jnp.concatenate([cb_ref[...]] * 4, axis=1)
        cbp = jnp.broadcast_to(cb4[:, None, :], (CCB, 8, 128))
        return gg_ref[...] - cbp.reshape(CH4, 128)

    def norm(yv, dst, grow, berow, lanes):
        xh = (yv - st_ref[dst:dst + 1, :lanes]) * st_ref[dst + 1:dst + 2,
                                                         :lanes]
        return jnp.maximum(xh * grow + berow, 0.0)

    def acc(yv, row, lanes):
        st_ref[row:row + 1, :lanes] += jnp.sum(yv, axis=0, keepdims=True)
        st_ref[row + 1:row + 2, :lanes] += jnp.sum(yv * yv, axis=0,
                                                   keepdims=True)

    @pl.when(p == 0)
    def _():
        acc(y1(), 0, 128)

    @pl.when(p == 1)
    def _():
        x1 = norm(y1(), 8, g1_ref[...], be1_ref[...], 128)
        y2 = lax.dot_general(x1, w2_ref[...], dn,
                             preferred_element_type=jnp.float32) + b2_ref[...]
        acc(y2, 2, 128)

    @pl.when(p == 2)
    def _():
        x1 = norm(y1(), 8, g1_ref[...], be1_ref[...], 128)
        y2 = lax.dot_general(x1, w2_ref[...], dn,
                             preferred_element_type=jnp.float32) + b2_ref[...]
        x2 = norm(y2, 10, g2_ref[...], be2_ref[...], 128)
        y3 = lax.dot_general(x2, w3_ref[...], dn,
                             preferred_element_type=jnp.float32) + b3_ref[...]
        acc(y3, 4, 256)

    @pl.when(p == 3)
    def _():
        x1 = norm(y1(), 8, g1_ref[...], be1_ref[...], 128)
        y2 = lax.dot_general(x1, w2_ref[...], dn,
                             preferred_element_type=jnp.float32) + b2_ref[...]
        x2 = norm(y2, 10, g2_ref[...], be2_ref[...], 128)
        y3 = lax.dot_general(x2, w3_ref[...], dn,
                             preferred_element_type=jnp.float32) + b3_ref[...]
        x3 = norm(y3, 12, g3_ref[...], be3_ref[...], 256)
        m8 = jnp.max(x3.reshape(CCB, 8, 256), axis=1)
        out_ref[...] = jnp.maximum(
            jnp.maximum(m8[:, 0:64], m8[:, 64:128]),
            jnp.maximum(m8[:, 128:192], m8[:, 192:256]))


def _mlp(ggp, cb_flat, w2b, b2p, w3b, b3p, g1p, be1p, g2p, be2p, g3p, be3p):
    wspec = lambda shp: pl.BlockSpec(shp, lambda p, t: (0, 0))
    return pl.pallas_call(
        _mlp_body,
        grid=(4, NCHUNK),
        in_specs=[
            pl.BlockSpec((CH4, 128), lambda p, t: (t, 0)),
            pl.BlockSpec((CCB, 32), lambda p, t: (t, 0)),
            wspec((128, 128)), wspec((1, 128)),
            wspec((128, 256)), wspec((1, 256)),
            wspec((1, 128)), wspec((1, 128)),
            wspec((1, 128)), wspec((1, 128)),
            wspec((1, 256)), wspec((1, 256)),
        ],
        out_specs=pl.BlockSpec((CCB, 64), lambda p, t: (t, 0)),
        out_shape=jax.ShapeDtypeStruct((B * S, 64), jnp.float32),
        scratch_shapes=[pltpu.VMEM((16, 256), jnp.float32)],
    )(ggp, cb_flat, w2b, b2p, w3b, b3p, g1p, be1p, g2p, be2p, g3p, be3p)


def kernel(xyz, points, W1, b1, gamma1, beta1, W2, b2, gamma2, beta2,
           W3, b3, gamma3, beta3):
    fps_raw = _fps(jnp.transpose(xyz, (1, 0, 2)))
    new_xyz = jnp.transpose(fps_raw, (1, 2, 0, 3)).reshape(B, 3, S)
    g, cb = _gtable(xyz, points, new_xyz, W1[:, :3], W1[:, 3:],
                    b1.reshape(1, 32))
    ball = _ballq(xyz.reshape(B * 3 * N), new_xyz.reshape(B * 3 * S))
    rows = _grows(g.reshape(B * N, 32), ball)
    eye4 = jnp.eye(4, dtype=jnp.float32)
    out_sp = _mlp(rows.reshape(B * S * K // 4, 128),
                  cb.reshape(B * S, 32),
                  jnp.kron(eye4, W2.T), jnp.tile(b2, 4).reshape(1, 128),
                  jnp.kron(eye4, W3.T), jnp.tile(b3, 4).reshape(1, 256),
                  jnp.tile(gamma1, 4).reshape(1, 128),
                  jnp.tile(beta1, 4).reshape(1, 128),
                  jnp.tile(gamma2, 4).reshape(1, 128),
                  jnp.tile(beta2, 4).reshape(1, 128),
                  jnp.tile(gamma3, 4).reshape(1, 256),
                  jnp.tile(beta3, 4).reshape(1, 256))
    new_points = jnp.transpose(out_sp.reshape(B, S, 64), (0, 2, 1))
    return (new_xyz, new_points)
